# baseline (speedup 1.0000x reference)
import functools

import jax
import jax.numpy as jnp
from jax import lax
from jax.experimental import pallas as pl
from jax.experimental.pallas import tpu as pltpu
from jax.experimental.pallas import tpu_sc as plsc

_NC = 2
_NS = 16
_NW = _NC * _NS
_BUF_ROWS = 56
_CHUNKS = (56, 56, 56, 56, 32)

_TC_CH = 4096
_TC_NBUF = 2


def _make_sc_copy(batch, seq_len, d_model):
    rows_per_w = seq_len // _NW
    assert sum(_CHUNKS) == rows_per_w
    nch = len(_CHUNKS)
    offs = [sum(_CHUNKS[:i]) for i in range(nch)]
    base = (batch - 1) * seq_len
    mesh = plsc.VectorSubcoreMesh(core_axis_name="c", subcore_axis_name="s")

    @functools.partial(
        pl.kernel,
        mesh=mesh,
        out_type=jax.ShapeDtypeStruct((batch * seq_len, d_model), jnp.float32),
        scratch_types=[
            pltpu.VMEM((_BUF_ROWS, d_model), jnp.float32),
            pltpu.VMEM((_BUF_ROWS, d_model), jnp.float32),
            pltpu.SemaphoreType.DMA,
            pltpu.SemaphoreType.DMA,
        ],
    )
    def sc_copy(table_hbm, out_hbm, buf0, buf1, insem, outsem):
        wid = lax.axis_index("s") * _NC + lax.axis_index("c")
        s0 = wid * rows_per_w
        bufs = (buf0, buf1)
        in_h = [None] * nch
        out_h = [None] * nch
        in_h[0] = pltpu.async_copy(
            table_hbm.at[pl.ds(s0, _CHUNKS[0])],
            buf0.at[pl.ds(0, _CHUNKS[0])],
            insem,
        )
        for c in range(nch):
            if c >= 1:
                out_h[c - 1].wait()
            if c + 1 < nch:
                in_h[c + 1] = pltpu.async_copy(
                    table_hbm.at[pl.ds(s0 + offs[c + 1], _CHUNKS[c + 1])],
                    bufs[(c + 1) % 2].at[pl.ds(0, _CHUNKS[c + 1])],
                    insem,
                )
            in_h[c].wait()
            out_h[c] = pltpu.async_copy(
                bufs[c % 2].at[pl.ds(0, _CHUNKS[c])],
                out_hbm.at[pl.ds(base + s0 + offs[c], _CHUNKS[c])],
                outsem,
            )
        out_h[nch - 1].wait()

    return sc_copy


def _make_tc_fill(batch, seq_len, d_model):
    nch = seq_len // _TC_CH
    nb = batch - 1

    def body(emb_hbm, prev_hbm, out_hbm, *rest):
        bufs = rest[:_TC_NBUF]
        insem, outsem = rest[_TC_NBUF], rest[_TC_NBUF + 1]
        in_h = [None] * nch
        out_h = [None] * nch
        in_h[0] = pltpu.make_async_copy(emb_hbm.at[pl.ds(0, _TC_CH)], bufs[0], insem)
        in_h[0].start()
        for c in range(nch):
            if c + 1 < nch:
                if c + 1 - _TC_NBUF >= 0:
                    for h in out_h[c + 1 - _TC_NBUF]:
                        h.wait()
                in_h[c + 1] = pltpu.make_async_copy(
                    emb_hbm.at[pl.ds((c + 1) * _TC_CH, _TC_CH)],
                    bufs[(c + 1) % _TC_NBUF],
                    insem,
                )
                in_h[c + 1].start()
            in_h[c].wait()
            buf = bufs[c % _TC_NBUF]
            out_h[c] = []
            for b in range(nb):
                h = pltpu.make_async_copy(
                    buf, out_hbm.at[pl.ds(b * seq_len + c * _TC_CH, _TC_CH)], outsem
                )
                h.start()
                out_h[c].append(h)
        for c in range(max(0, nch - _TC_NBUF), nch):
            for h in out_h[c]:
                h.wait()

    return pl.pallas_call(
        body,
        in_specs=[
            pl.BlockSpec(memory_space=pl.ANY),
            pl.BlockSpec(memory_space=pl.ANY),
        ],
        out_specs=pl.BlockSpec(memory_space=pl.ANY),
        out_shape=jax.ShapeDtypeStruct((batch * seq_len, d_model), jnp.float32),
        input_output_aliases={1: 0},
        scratch_shapes=[
            pltpu.VMEM((_TC_CH, d_model), jnp.float32) for _ in range(_TC_NBUF)
        ]
        + [pltpu.SemaphoreType.DMA, pltpu.SemaphoreType.DMA],
    )


def kernel(x, pos_embedding):
    batch, seq_len = x.shape
    max_len, d_model = pos_embedding.shape
    sc_out = _make_sc_copy(batch, seq_len, d_model)(pos_embedding)
    out_flat = _make_tc_fill(batch, seq_len, d_model)(pos_embedding, sc_out)
    return out_flat.reshape(batch, seq_len, d_model)


# speedup vs baseline: 1.1045x; 1.1045x over previous
import functools

import jax
import jax.numpy as jnp
from jax import lax
from jax.experimental import pallas as pl
from jax.experimental.pallas import tpu as pltpu
from jax.experimental.pallas import tpu_sc as plsc

_NC = 2
_NS = 16
_NW = _NC * _NS
_BUF_ROWS = 56
_CHUNKS = (56, 56, 56, 56, 32)


def _make_sc_copy(batch, seq_len, d_model):
    rows_per_w = seq_len // _NW
    assert sum(_CHUNKS) == rows_per_w
    nch = len(_CHUNKS)
    offs = [sum(_CHUNKS[:i]) for i in range(nch)]
    mesh = plsc.VectorSubcoreMesh(core_axis_name="c", subcore_axis_name="s")

    @functools.partial(
        pl.kernel,
        mesh=mesh,
        out_type=jax.ShapeDtypeStruct((batch * seq_len, d_model), jnp.float32),
        scratch_types=[
            pltpu.VMEM((_BUF_ROWS, d_model), jnp.float32),
            pltpu.VMEM((_BUF_ROWS, d_model), jnp.float32),
            pltpu.SemaphoreType.DMA,
            pltpu.SemaphoreType.DMA,
        ],
    )
    def sc_copy(table_hbm, out_hbm, buf0, buf1, insem, outsem):
        wid = lax.axis_index("s") * _NC + lax.axis_index("c")
        s0 = wid * rows_per_w
        bufs = (buf0, buf1)
        in_h = [None] * nch
        out_h = [None] * nch
        in_h[0] = pltpu.async_copy(
            table_hbm.at[pl.ds(s0, _CHUNKS[0])],
            buf0.at[pl.ds(0, _CHUNKS[0])],
            insem,
        )
        for c in range(nch):
            if c >= 1:
                for h in out_h[c - 1]:
                    h.wait()
            if c + 1 < nch:
                in_h[c + 1] = pltpu.async_copy(
                    table_hbm.at[pl.ds(s0 + offs[c + 1], _CHUNKS[c + 1])],
                    bufs[(c + 1) % 2].at[pl.ds(0, _CHUNKS[c + 1])],
                    insem,
                )
            in_h[c].wait()
            buf = bufs[c % 2]
            out_h[c] = [
                pltpu.async_copy(
                    buf.at[pl.ds(0, _CHUNKS[c])],
                    out_hbm.at[pl.ds(b * seq_len + s0 + offs[c], _CHUNKS[c])],
                    outsem,
                )
                for b in range(batch)
            ]
        for h in out_h[nch - 1]:
            h.wait()

    return sc_copy


def kernel(x, pos_embedding):
    batch, seq_len = x.shape
    max_len, d_model = pos_embedding.shape
    out_flat = _make_sc_copy(batch, seq_len, d_model)(pos_embedding)
    return out_flat.reshape(batch, seq_len, d_model)
